# baseline (speedup 1.0000x reference)
import functools

import jax
import jax.numpy as jnp
from jax import lax
from jax.experimental import pallas as pl
from jax.experimental.pallas import tpu as pltpu
from jax.experimental.pallas import tpu_sc as plsc

N_NODES = 10000
D = 128
E_EDGES = 320000

NUM_CORES = 2
NUM_SUBCORES = 16
NW = NUM_CORES * NUM_SUBCORES
CHUNK = 32
NBUF = 8
CHUNKS_PER_W = 320
E_PAD = NW * CHUNKS_PER_W * CHUNK
N_PAD = 10112
ROWS_PER_SUB = N_PAD // NUM_SUBCORES


def _mm_body(x_ref, wt_ref, b_ref, o_ref):
    o_ref[...] = (
        jnp.dot(x_ref[...], wt_ref[...], preferred_element_type=jnp.float32)
        + b_ref[...]
    )


def _mm(x, wt, b):
    blk = 1000
    return pl.pallas_call(
        _mm_body,
        grid=(N_NODES // blk,),
        in_specs=[
            pl.BlockSpec((blk, D), lambda i: (i, 0)),
            pl.BlockSpec((D, D), lambda i: (0, 0)),
            pl.BlockSpec((1, D), lambda i: (0, 0)),
        ],
        out_specs=pl.BlockSpec((blk, D), lambda i: (i, 0)),
        out_shape=jax.ShapeDtypeStruct((N_NODES, D), jnp.float32),
    )(x, wt, b.reshape(1, D))


def _mm_relu_sum_body(a_ref, b2_ref, wt_ref, b_ref, o_ref):
    h = jnp.maximum(a_ref[...] + b2_ref[...], 0.0)
    o_ref[...] = (
        jnp.dot(h, wt_ref[...], preferred_element_type=jnp.float32) + b_ref[...]
    )


def _mm_relu_sum(ya, yb, wt, b):
    blk = 1000
    return pl.pallas_call(
        _mm_relu_sum_body,
        grid=(N_NODES // blk,),
        in_specs=[
            pl.BlockSpec((blk, D), lambda i: (i, 0)),
            pl.BlockSpec((blk, D), lambda i: (i, 0)),
            pl.BlockSpec((D, D), lambda i: (0, 0)),
            pl.BlockSpec((1, D), lambda i: (0, 0)),
        ],
        out_specs=pl.BlockSpec((blk, D), lambda i: (i, 0)),
        out_shape=jax.ShapeDtypeStruct((N_NODES, D), jnp.float32),
    )(ya, yb, wt, b.reshape(1, D))


def _add_body(a_ref, b_ref, o_ref):
    o_ref[...] = a_ref[...] + b_ref[...]


def _add(a, b):
    blk = 1000
    return pl.pallas_call(
        _add_body,
        grid=(N_NODES // blk,),
        in_specs=[
            pl.BlockSpec((blk, D), lambda i: (i, 0)),
            pl.BlockSpec((blk, D), lambda i: (i, 0)),
        ],
        out_specs=pl.BlockSpec((blk, D), lambda i: (i, 0)),
        out_shape=jax.ShapeDtypeStruct((N_NODES, D), jnp.float32),
    )(a, b)


def _scale_chunk(rows_v, w_v):
    def group_body(g, _):
        gb = g * 16
        for j in range(16):
            e = gb + j
            wbc = w_v[pl.ds(e * 16, 16)]
            for f in range(D // 16):
                sl = pl.ds(f * 16, 16)
                rows_v[e, sl] = rows_v[e, sl] * wbc
        return 0

    lax.fori_loop(0, CHUNK // 16, group_body, 0)


def _scatter_kernel(h_hbm, sd_hbm, w_hbm, out_hbm,
                    sds, wvs, rows, acc_sh, si, sg, ss):
    c = lax.axis_index("c")
    s = lax.axis_index("s")
    wid = c * NUM_SUBCORES + s
    base = wid * CHUNKS_PER_W

    zeros16 = jnp.zeros((16,), jnp.float32)
    r0 = rows[0]

    def zero_row(r, _):
        for f in range(D // 16):
            r0[r, pl.ds(f * 16, 16)] = zeros16
        return 0

    lax.fori_loop(0, CHUNK, zero_row, 0)
    rbase = s * ROWS_PER_SUB
    for off in range(0, ROWS_PER_SUB, CHUNK):
        n = min(CHUNK, ROWS_PER_SUB - off)
        pltpu.sync_copy(r0.at[pl.ds(0, n)],
                        acc_sh.at[pl.ds(rbase + off, n)])
    plsc.subcore_barrier()

    def group_body(g, _):
        i0 = g * NBUF
        ins = []
        for k in range(NBUF):
            ins.append(pltpu.async_copy(sd_hbm.at[base + i0 + k], sds[k], si))
            ins.append(pltpu.async_copy(
                w_hbm.at[pl.ds((base + i0 + k) * CHUNK * 16, CHUNK * 16)],
                wvs[k], si))
        for a in ins:
            a.wait()
        gs = [pltpu.async_copy(h_hbm.at[sds[k].at[0]], rows[k], sg)
              for k in range(NBUF)]
        for a in gs:
            a.wait()
        for k in range(NBUF):
            _scale_chunk(rows[k], wvs[k])
        scs = [pltpu.async_copy(rows[k], acc_sh.at[sds[k].at[1]], ss,
                                add=True)
               for k in range(NBUF)]
        for a in scs:
            a.wait()
        return 0

    lax.fori_loop(0, CHUNKS_PER_W // NBUF, group_body, 0)
    plsc.subcore_barrier()

    pltpu.sync_copy(acc_sh.at[pl.ds(rbase, ROWS_PER_SUB)],
                    out_hbm.at[c, pl.ds(rbase, ROWS_PER_SUB)])


def _scatter_wrapped(h, sd, w):
    def body(h_hbm, sd_hbm, w_hbm, out_hbm, *scr):
        sds = scr[:NBUF]
        wvs = scr[NBUF:2 * NBUF]
        rows = scr[2 * NBUF:3 * NBUF]
        acc_sh = scr[3 * NBUF]
        si, sg, ss = scr[3 * NBUF + 1:3 * NBUF + 4]
        _scatter_kernel(h_hbm, sd_hbm, w_hbm, out_hbm,
                        sds, wvs, rows, acc_sh, si, sg, ss)

    scratch = (
        [pltpu.VMEM((2, CHUNK), jnp.int32)] * NBUF
        + [pltpu.VMEM((CHUNK * 16,), jnp.float32)] * NBUF
        + [pltpu.VMEM((CHUNK, D), jnp.float32)] * NBUF
        + [pltpu.VMEM_SHARED((N_PAD, D), jnp.float32)]
        + [pltpu.SemaphoreType.DMA] * 3
    )
    return pl.kernel(
        body,
        out_type=jax.ShapeDtypeStruct((NUM_CORES, N_PAD, D), jnp.float32),
        mesh=plsc.VectorSubcoreMesh(core_axis_name="c", subcore_axis_name="s"),
        scratch_types=scratch,
    )(h, sd, w)


_scatter = _scatter_wrapped


def _pad1d(a, fill):
    pad = E_PAD - E_EDGES
    return jnp.concatenate([a, jnp.full((pad,), fill, a.dtype)])


def _pad2d(a, fill):
    return _pad1d(a, fill).reshape(-1, CHUNK)


def kernel(x, edge_index, w0, w1, W0, b0, W1, b1):
    src = _pad2d(edge_index[0].astype(jnp.int32), 0)
    dst = _pad2d(edge_index[1].astype(jnp.int32), 0)
    sd = jnp.stack([src, dst], axis=1)
    w0p = jnp.broadcast_to(_pad1d(w0.astype(jnp.float32), 0.0)[:, None],
                           (E_PAD, 16)).reshape(-1)
    w1p = jnp.broadcast_to(_pad1d(w1.astype(jnp.float32), 0.0)[:, None],
                           (E_PAD, 16)).reshape(-1)

    h0 = _mm(x, W0.T, b0)
    y0 = _scatter(h0, sd, w0p)
    h1 = _mm_relu_sum(y0[0, :N_NODES], y0[1, :N_NODES], W1.T, b1)
    y1 = _scatter(h1, sd, w1p)
    return _add(y1[0, :N_NODES], y1[1, :N_NODES])


# speedup vs baseline: 1.1838x; 1.1838x over previous
import functools

import jax
import jax.numpy as jnp
from jax import lax
from jax.experimental import pallas as pl
from jax.experimental.pallas import tpu as pltpu
from jax.experimental.pallas import tpu_sc as plsc

N_NODES = 10000
D = 128
E_EDGES = 320000

NUM_CORES = 2
NUM_SUBCORES = 16
NW = NUM_CORES * NUM_SUBCORES
CHUNK = 32
NBUF = 8
CHUNKS_PER_W = 320
E_PAD = NW * CHUNKS_PER_W * CHUNK
N_PAD = 10112
ROWS_PER_SUB = N_PAD // NUM_SUBCORES


def _mm_body(x_ref, wt_ref, b_ref, o_ref):
    o_ref[...] = (
        jnp.dot(x_ref[...], wt_ref[...], preferred_element_type=jnp.float32)
        + b_ref[...]
    )


def _mm(x, wt, b):
    blk = 1000
    return pl.pallas_call(
        _mm_body,
        grid=(N_NODES // blk,),
        in_specs=[
            pl.BlockSpec((blk, D), lambda i: (i, 0)),
            pl.BlockSpec((D, D), lambda i: (0, 0)),
            pl.BlockSpec((1, D), lambda i: (0, 0)),
        ],
        out_specs=pl.BlockSpec((blk, D), lambda i: (i, 0)),
        out_shape=jax.ShapeDtypeStruct((N_NODES, D), jnp.float32),
    )(x, wt, b.reshape(1, D))


def _mm_relu_sum_body(a_ref, b2_ref, wt_ref, b_ref, o_ref):
    h = jnp.maximum(a_ref[...] + b2_ref[...], 0.0)
    o_ref[...] = (
        jnp.dot(h, wt_ref[...], preferred_element_type=jnp.float32) + b_ref[...]
    )


def _mm_relu_sum(ya, yb, wt, b):
    blk = 1000
    return pl.pallas_call(
        _mm_relu_sum_body,
        grid=(N_NODES // blk,),
        in_specs=[
            pl.BlockSpec((blk, D), lambda i: (i, 0)),
            pl.BlockSpec((blk, D), lambda i: (i, 0)),
            pl.BlockSpec((D, D), lambda i: (0, 0)),
            pl.BlockSpec((1, D), lambda i: (0, 0)),
        ],
        out_specs=pl.BlockSpec((blk, D), lambda i: (i, 0)),
        out_shape=jax.ShapeDtypeStruct((N_NODES, D), jnp.float32),
    )(ya, yb, wt, b.reshape(1, D))


def _add_body(a_ref, b_ref, o_ref):
    o_ref[...] = a_ref[...] + b_ref[...]


def _add(a, b):
    blk = 1000
    return pl.pallas_call(
        _add_body,
        grid=(N_NODES // blk,),
        in_specs=[
            pl.BlockSpec((blk, D), lambda i: (i, 0)),
            pl.BlockSpec((blk, D), lambda i: (i, 0)),
        ],
        out_specs=pl.BlockSpec((blk, D), lambda i: (i, 0)),
        out_shape=jax.ShapeDtypeStruct((N_NODES, D), jnp.float32),
    )(a, b)


def _scale_chunk(rows_v, w_v):
    def group_body(g, _):
        gb = g * 16
        for j in range(16):
            e = gb + j
            wbc = w_v[pl.ds(e * 16, 16)]
            for f in range(D // 16):
                sl = pl.ds(f * 16, 16)
                rows_v[e, sl] = rows_v[e, sl] * wbc
        return 0

    lax.fori_loop(0, CHUNK // 16, group_body, 0)


def _scatter_kernel(h_hbm, sd_hbm, w_hbm, out_hbm,
                    sds, wvs, rows, acc_sh, si, sg, ss):
    c = lax.axis_index("c")
    s = lax.axis_index("s")
    wid = c * NUM_SUBCORES + s
    base = wid * CHUNKS_PER_W

    zeros16 = jnp.zeros((16,), jnp.float32)
    r0 = rows[0]

    def zero_row(r, _):
        for f in range(D // 16):
            r0[r, pl.ds(f * 16, 16)] = zeros16
        return 0

    lax.fori_loop(0, CHUNK, zero_row, 0)
    rbase = s * ROWS_PER_SUB
    for off in range(0, ROWS_PER_SUB, CHUNK):
        n = min(CHUNK, ROWS_PER_SUB - off)
        pltpu.sync_copy(r0.at[pl.ds(0, n)],
                        acc_sh.at[pl.ds(rbase + off, n)])

    GSZ = NBUF // 2
    NG = CHUNKS_PER_W // GSZ

    def ins_of(g, half):
        out = []
        for k in range(GSZ):
            b = half * GSZ + k
            i = g * GSZ + k
            out.append((sd_hbm.at[base + i], sds[b], si[half]))
            out.append((w_hbm.at[pl.ds((base + i) * CHUNK * 16, CHUNK * 16)],
                        wvs[b], si[half]))
        return out

    def fire_ins(g, half):
        for a in ins_of(g, half):
            pltpu.async_copy(*a)

    def wait_ins(g, half):
        for a in ins_of(g, half):
            pltpu.make_async_copy(*a).wait()

    def gathers_of(half):
        return [(h_hbm.at[sds[half * GSZ + k].at[0]], rows[half * GSZ + k],
                 sg[half]) for k in range(GSZ)]

    def fire_gathers(half):
        for a in gathers_of(half):
            pltpu.async_copy(*a)

    def wait_gathers(half):
        for a in gathers_of(half):
            pltpu.make_async_copy(*a).wait()

    def scatters_of(half):
        return [(rows[half * GSZ + k], acc_sh.at[sds[half * GSZ + k].at[1]],
                 ss[half]) for k in range(GSZ)]

    def fire_scatters(half):
        for a in scatters_of(half):
            pltpu.async_copy(*a, add=True)

    def wait_scatters(half):
        for a in scatters_of(half):
            pltpu.make_async_copy(*a).wait()

    def stage(g, p, half):
        wait_gathers(half)
        for k in range(GSZ):
            _scale_chunk(rows[half * GSZ + k], wvs[half * GSZ + k])
        fire_scatters(half)

        @pl.when(p < NG // 2 - 1)
        def _():
            wait_scatters(half)
            fire_ins(g + 2, half)
            wait_ins(g + 2, half)
            fire_gathers(half)

    fire_ins(0, 0)
    wait_ins(0, 0)
    fire_gathers(0)
    fire_ins(1, 1)
    wait_ins(1, 1)
    fire_gathers(1)
    plsc.subcore_barrier()

    def pair_body(p, _):
        stage(2 * p, p, 0)
        stage(2 * p + 1, p, 1)
        return 0

    lax.fori_loop(0, NG // 2, pair_body, 0)
    wait_scatters(0)
    wait_scatters(1)
    plsc.subcore_barrier()

    pltpu.sync_copy(acc_sh.at[pl.ds(rbase, ROWS_PER_SUB)],
                    out_hbm.at[c, pl.ds(rbase, ROWS_PER_SUB)])


def _scatter_wrapped(h, sd, w):
    def body(h_hbm, sd_hbm, w_hbm, out_hbm, *scr):
        sds = scr[:NBUF]
        wvs = scr[NBUF:2 * NBUF]
        rows = scr[2 * NBUF:3 * NBUF]
        acc_sh = scr[3 * NBUF]
        si = scr[3 * NBUF + 1:3 * NBUF + 3]
        sg = scr[3 * NBUF + 3:3 * NBUF + 5]
        ss = scr[3 * NBUF + 5:3 * NBUF + 7]
        _scatter_kernel(h_hbm, sd_hbm, w_hbm, out_hbm,
                        sds, wvs, rows, acc_sh, si, sg, ss)

    scratch = (
        [pltpu.VMEM((2, CHUNK), jnp.int32)] * NBUF
        + [pltpu.VMEM((CHUNK * 16,), jnp.float32)] * NBUF
        + [pltpu.VMEM((CHUNK, D), jnp.float32)] * NBUF
        + [pltpu.VMEM_SHARED((N_PAD, D), jnp.float32)]
        + [pltpu.SemaphoreType.DMA] * 6
    )
    return pl.kernel(
        body,
        out_type=jax.ShapeDtypeStruct((NUM_CORES, N_PAD, D), jnp.float32),
        mesh=plsc.VectorSubcoreMesh(core_axis_name="c", subcore_axis_name="s"),
        scratch_types=scratch,
    )(h, sd, w)


_scatter = _scatter_wrapped


def _pad1d(a, fill):
    pad = E_PAD - E_EDGES
    return jnp.concatenate([a, jnp.full((pad,), fill, a.dtype)])


def _pad2d(a, fill):
    return _pad1d(a, fill).reshape(-1, CHUNK)


def kernel(x, edge_index, w0, w1, W0, b0, W1, b1):
    src = _pad2d(edge_index[0].astype(jnp.int32), 0)
    dst = _pad2d(edge_index[1].astype(jnp.int32), 0)
    sd = jnp.stack([src, dst], axis=1)
    w0p = jnp.broadcast_to(_pad1d(w0.astype(jnp.float32), 0.0)[:, None],
                           (E_PAD, 16)).reshape(-1)
    w1p = jnp.broadcast_to(_pad1d(w1.astype(jnp.float32), 0.0)[:, None],
                           (E_PAD, 16)).reshape(-1)

    h0 = _mm(x, W0.T, b0)
    y0 = _scatter(h0, sd, w0p)
    h1 = _mm_relu_sum(y0[0, :N_NODES], y0[1, :N_NODES], W1.T, b1)
    y1 = _scatter(h1, sd, w1p)
    return _add(y1[0, :N_NODES], y1[1, :N_NODES])
